# baseline (speedup 1.0000x reference)
import dataclasses
import functools

import jax
import jax.numpy as jnp
from jax import lax
from jax.experimental import pallas as pl
from jax.experimental.pallas import tpu as pltpu
from jax.experimental.pallas import tpu_sc as plsc

D = 128
L = 16
NC, NS = 2, 16
NW = NC * NS
C = 128


@functools.cache
def _dot_kernel(E_pad, n_rows):
    per_w = E_pad // NW
    n_chunks = per_w // C
    assert n_chunks % 2 == 0
    del n_rows

    mesh = plsc.VectorSubcoreMesh(core_axis_name="c", subcore_axis_name="s")

    cp = pltpu.CompilerParams()
    if "needs_layout_passes" in pltpu.CompilerParams.__dataclass_fields__:
        cp = dataclasses.replace(cp, needs_layout_passes=False)

    @functools.partial(
        pl.kernel,
        mesh=mesh,
        compiler_params=cp,
        out_type=jax.ShapeDtypeStruct((E_pad,), jnp.float32),
        scratch_types=[
            pltpu.VMEM((n_chunks, C), jnp.int32),
            pltpu.VMEM((n_chunks, C), jnp.int32),
            pltpu.VMEM((C, D), jnp.float32),
            pltpu.VMEM((C, D), jnp.float32),
            pltpu.VMEM((C, D), jnp.float32),
            pltpu.VMEM((C, D), jnp.float32),
            pltpu.VMEM((per_w,), jnp.float32),
            pltpu.SemaphoreType.DMA,
            pltpu.SemaphoreType.DMA,
            pltpu.SemaphoreType.DMA,
        ],
    )
    def k(h_hbm, src_hbm, dst_hbm, out_hbm,
          sidx, didx, srows0, drows0, srows1, drows1, ovec,
          sem_i, sem_g0, sem_g1):
        cid = lax.axis_index("c")
        sid = lax.axis_index("s")
        wid = sid * NC + cid
        base = wid * per_w

        ci = pltpu.async_copy(src_hbm.at[wid], sidx, sem_i)
        ci2 = pltpu.async_copy(dst_hbm.at[wid], didx, sem_i)
        ci.wait()
        ci2.wait()

        bufs = ((srows0, drows0, sem_g0), (srows1, drows1, sem_g1))

        def issue(t, b):
            srb, drb, sem = bufs[b]
            pltpu.async_copy(h_hbm.at[sidx.at[t]], srb, sem)
            pltpu.async_copy(h_hbm.at[didx.at[t]], drb, sem)

        def wait_gathers(b):
            srb, drb, sem = bufs[b]
            pltpu.make_async_copy(h_hbm.at[pl.ds(0, C)], srb, sem).wait()
            pltpu.make_async_copy(h_hbm.at[pl.ds(0, C)], drb, sem).wait()

        def compute(t, b):
            srb, drb, _ = bufs[b]
            lane = lax.iota(jnp.int32, L)
            last = lane == (L - 1)

            @pl.loop(0, C // L)
            def _grp(g):
                o0 = t * C + g * L
                o0v = jnp.full((L,), 0, jnp.int32) + o0
                for j in range(L):
                    e = g * L + j
                    p = srb[e, pl.ds(0, L)] * drb[e, pl.ds(0, L)]
                    for kk in range(1, D // L):
                        p = p + (srb[e, pl.ds(kk * L, L)]
                                 * drb[e, pl.ds(kk * L, L)])
                    ps = lax.cumsum(p, axis=0)
                    plsc.store_scatter(ovec, [o0v + j], ps, mask=last)

        issue(0, 0)
        issue(1, 1)

        @pl.loop(0, n_chunks, step=2)
        def _chunk(t):
            wait_gathers(0)
            compute(t, 0)

            @pl.when(t + 2 < n_chunks)
            def _():
                issue(t + 2, 0)

            wait_gathers(1)
            compute(t + 1, 1)

            @pl.when(t + 3 < n_chunks)
            def _():
                issue(t + 3, 1)

        pltpu.sync_copy(ovec, out_hbm.at[pl.ds(base, per_w)])

    return k


def kernel(h, edge_index):
    E = edge_index.shape[1]
    src = edge_index[0].astype(jnp.int32)
    dst = edge_index[1].astype(jnp.int32)

    step = NW * C * 2
    E_pad = ((E + step - 1) // step) * step
    if E_pad != E:
        pad = E_pad - E
        zeros = jnp.zeros((pad,), jnp.int32)
        src = jnp.concatenate([src, zeros])
        dst = jnp.concatenate([dst, zeros])

    per_w = E_pad // NW
    src = src.reshape(NW, per_w // C, C)
    dst = dst.reshape(NW, per_w // C, C)

    out = _dot_kernel(E_pad, h.shape[0])(h, src, dst)
    return out[:E].reshape(E, 1)
